# baseline (speedup 1.0000x reference)
import jax
import jax.numpy as jnp
from jax.experimental import pallas as pl

_SUB = 4096


def _distnet_kernel(xt_ref, ptst_ref, beta_ref, out_ref):
    xt = xt_ref[...]
    x = xt.T
    pts_t = ptst_ref[...]
    ptf = pts_t.astype(jnp.float32)
    pp = jnp.sum(ptf * ptf, axis=0, keepdims=True)
    lhs = jnp.concatenate(
        [jnp.ones((x.shape[0], 1), jnp.bfloat16), x.astype(jnp.bfloat16)],
        axis=1)
    rhs = jnp.concatenate(
        [pp.astype(jnp.bfloat16), -2.0 * pts_t], axis=0)
    n = ptst_ref.shape[1]
    mblk = None
    prev = None
    for s in range(0, n, _SUB):
        partial = jax.lax.dot_general(
            lhs, rhs[:, s:min(s + _SUB, n)], (((1,), (0,)), ((), ())),
            preferred_element_type=jnp.float32)
        if prev is not None:
            m = jnp.min(prev, axis=1, keepdims=True)
            mblk = m if mblk is None else jnp.minimum(mblk, m)
        prev = partial
    m = jnp.min(prev, axis=1, keepdims=True)
    mblk = m if mblk is None else jnp.minimum(mblk, m)

    xx = jnp.sum(x * x, axis=1, keepdims=True)
    d = jnp.maximum(mblk + xx, 0.0)
    b = jax.nn.softplus(beta_ref[0, 0])
    alpha = -b * 6.9077542789816375
    out_ref[...] = jax.nn.sigmoid((d + alpha) / b)


def kernel(x, points, beta):
    q, dim = x.shape
    n_pts = points.shape[0]
    pts_t = points.T.astype(jnp.bfloat16)
    xt = x.T
    beta2d = beta.reshape(1, 1)
    out = pl.pallas_call(
        _distnet_kernel,
        grid=(1,),
        in_specs=[
            pl.BlockSpec((dim, q), lambda i: (0, 0)),
            pl.BlockSpec((dim, n_pts), lambda i: (0, i)),
            pl.BlockSpec((1, 1), lambda i: (0, 0)),
        ],
        out_specs=pl.BlockSpec((q, 1), lambda i: (0, 0)),
        out_shape=jax.ShapeDtypeStruct((q, 1), jnp.float32),
    )(xt, pts_t, beta2d)
    return out.reshape(q)
